# baseline (speedup 1.0000x reference)
import functools

import jax
import jax.numpy as jnp
from jax import lax
from jax.experimental import pallas as pl
from jax.experimental.pallas import tpu as pltpu
from jax.experimental.pallas import tpu_sc as plsc

_NROWS = 128
_NCOLS = 8192
_NZEROS = _NCOLS - 4096
_SCALE = 0.8 / (2.0 / 3.0)

_NCHUNK = _NCOLS // 16


def _sc_body(z_hbm, out_hbm, in_v, out_v, hist_v):
    nc = 2
    wid = lax.axis_index("s") * nc + lax.axis_index("c")
    rows_per_w = _NROWS // 32

    zero16 = jnp.zeros((16,), jnp.int32)
    ones16 = jnp.ones((16,), jnp.int32)
    iota16 = lax.iota(jnp.int32, 16)

    def _zero_hist(j, c):
        hist_v[pl.ds(j * 16, 16)] = zero16
        return c

    lax.fori_loop(0, 128, _zero_hist, 0)

    def _keys(i):
        zc = in_v[pl.ds(i * 16, 16)]
        y = lax.bitcast_convert_type(zc, jnp.int32)
        v = jnp.where(y < 0, ~y, y ^ jnp.int32(-(2**31)))
        return zc, lax.bitcast_convert_type(v, jnp.uint32)

    def _scan_bins(nbins, k):
        def sb(j, carry):
            found, bsel, before, run = carry
            h = hist_v[pl.ds(j * 16, 16)]
            hist_v[pl.ds(j * 16, 16)] = zero16
            c = plsc.cumsum(h)
            tot = run + c
            crossed = tot >= k
            ci = jnp.where(crossed, ones16, zero16)
            first = crossed & (plsc.cumsum(ci) == 1)
            hit = jnp.sum(ci) > 0
            newb = j * 16 + jnp.sum(jnp.where(first, iota16, zero16))
            newbefore = run + jnp.sum(jnp.where(first, c - h, zero16))
            take = hit & jnp.logical_not(found)
            bsel = jnp.where(take, newb, bsel)
            before = jnp.where(take, newbefore, before)
            found = found | hit
            run = run + jnp.sum(h)
            return found, bsel, before, run

        init = (jnp.bool_(False), jnp.int32(0), jnp.int32(0), jnp.int32(0))
        _, b, before, _ = lax.fori_loop(0, nbins // 16, sb, init)
        return b, before

    def _row_body(r, c):
        row = wid * rows_per_w + r
        pltpu.sync_copy(z_hbm.at[row], in_v)

        k = jnp.int32(_NZEROS)

        def h1(i, cc):
            _, u = _keys(i)
            b = (u >> jnp.uint32(21)).astype(jnp.int32)
            plsc.addupdate_scatter(hist_v, [b], ones16)
            return cc

        lax.fori_loop(0, _NCHUNK, h1, 0)
        b1, before1 = _scan_bins(2048, k)
        b1u = b1.astype(jnp.uint32)

        def h2(i, cc):
            _, u = _keys(i)
            match = (u >> jnp.uint32(21)) == b1u
            b = ((u >> jnp.uint32(10)) & jnp.uint32(0x7FF)).astype(jnp.int32)
            plsc.addupdate_scatter(hist_v, [b], ones16, mask=match)
            return cc

        lax.fori_loop(0, _NCHUNK, h2, 0)
        b2, before2 = _scan_bins(2048, k - before1)
        pref2 = (b1u << jnp.uint32(11)) | b2.astype(jnp.uint32)

        def h3(i, cc):
            _, u = _keys(i)
            match = (u >> jnp.uint32(10)) == pref2
            b = (u & jnp.uint32(0x3FF)).astype(jnp.int32)
            plsc.addupdate_scatter(hist_v, [b], ones16, mask=match)
            return cc

        lax.fori_loop(0, _NCHUNK, h3, 0)
        b3, before3 = _scan_bins(1024, k - before1 - before2)

        t_u = (pref2 << jnp.uint32(10)) | b3.astype(jnp.uint32)
        need = k - before1 - before2 - before3

        def outp(i, run_eq):
            zc, u = _keys(i)
            lt = u < t_u
            eq = u == t_u
            eqi = jnp.where(eq, ones16, zero16)
            rank = run_eq + plsc.cumsum(eqi) - eqi
            zeroed = lt | (eq & (rank < need))
            s = 1.0 / (1.0 + jnp.exp(zc * jnp.float32(-_SCALE)))
            out_v[pl.ds(i * 16, 16)] = jnp.where(zeroed, jnp.float32(0.0), s)
            return run_eq + jnp.sum(eqi)

        lax.fori_loop(0, _NCHUNK, outp, jnp.int32(0))

        pltpu.sync_copy(out_v, out_hbm.at[row])
        return c

    lax.fori_loop(0, rows_per_w, _row_body, 0)


@jax.jit
def kernel(z_loga):
    mesh = plsc.VectorSubcoreMesh(core_axis_name="c", subcore_axis_name="s")
    f = functools.partial(
        pl.kernel,
        mesh=mesh,
        out_type=jax.ShapeDtypeStruct((_NROWS, _NCOLS), jnp.float32),
        scratch_types=[
            pltpu.VMEM((_NCOLS,), jnp.float32),
            pltpu.VMEM((_NCOLS,), jnp.float32),
            pltpu.VMEM((2048,), jnp.int32),
        ],
        compiler_params=pltpu.CompilerParams(needs_layout_passes=False),
    )(_sc_body)
    return f(z_loga)
